# baseline (speedup 1.0000x reference)
import functools

import jax
import jax.numpy as jnp
from jax import lax
from jax.experimental import pallas as pl
from jax.experimental.pallas import tpu as pltpu
from jax.experimental.pallas import tpu_sc as plsc

_D = 9
_RAD = 4
_YRES = 256
_XRES = 256
_H = _YRES * _D
_W = _XRES * _D
_C = 3
_ROWS = _C * _H
_NW = 32
_RPW = _ROWS // _NW
_NB = 8
_NBLK = _RPW // _NB
_L = 16
_JV = _W // _L


def _coeffs(alpha, n_res):
    idx = jnp.arange(n_res * _D)
    mp = idx // _D
    off = idx % _D
    d = (off - _RAD).astype(jnp.float32)
    ind = mp.astype(jnp.float32) - alpha * d
    fl = jnp.floor(ind)
    w2 = ind - fl
    w1 = 1.0 - w2
    g1 = jnp.clip(fl, 0, n_res - 1).astype(jnp.int32)
    g2 = jnp.clip(fl + 1.0, 0, n_res - 1).astype(jnp.int32)
    r1 = off + g1 * _D
    r2 = off + g2 * _D
    return r1, r2, w1, w2


def _sc_body(x_hbm, rows1_hbm, rows2_hbm, wf_hbm,
             c1_hbm, c2_hbm, b1_hbm, out_hbm,
             idx1_v, idx2_v, wf_v, abuf, bbuf, obuf,
             c1_v, c2_v, b1_v, sem_a, sem_b, sem_o):
    wid = lax.axis_index("s") * 2 + lax.axis_index("c")
    my_base = wid * _RPW

    pltpu.sync_copy(c1_hbm, c1_v)
    pltpu.sync_copy(c2_hbm, c2_v)
    pltpu.sync_copy(b1_hbm, b1_v)
    pltpu.sync_copy(rows1_hbm.at[pl.ds(my_base, _RPW)], idx1_v)
    pltpu.sync_copy(rows2_hbm.at[pl.ds(my_base, _RPW)], idx2_v)
    pltpu.sync_copy(wf_hbm.at[pl.ds(my_base, _RPW)], wf_v)

    def stage(blk, s):
        ofs = blk * _NB
        pltpu.async_copy(
            x_hbm.at[idx1_v.at[pl.ds(ofs, _NB)]], abuf.at[s], sem_a.at[s])
        pltpu.async_copy(
            x_hbm.at[idx2_v.at[pl.ds(ofs, _NB)]], bbuf.at[s], sem_b.at[s])

    def gwait(s):
        pltpu.make_async_copy(
            x_hbm.at[idx1_v.at[pl.ds(0, _NB)]], abuf.at[s], sem_a.at[s]).wait()
        pltpu.make_async_copy(
            x_hbm.at[idx2_v.at[pl.ds(0, _NB)]], bbuf.at[s], sem_b.at[s]).wait()

    def store(blk, s):
        base = blk * _NB + my_base
        pltpu.async_copy(
            obuf.at[s], out_hbm.at[pl.ds(base, _NB)], sem_o.at[s])

    def swait(blk, s):
        base = blk * _NB + my_base
        pltpu.make_async_copy(
            obuf.at[s], out_hbm.at[pl.ds(base, _NB)], sem_o.at[s]).wait()

    def compute(blk, s):
        ofs = blk * _NB
        w0 = tuple(wf_v[ofs + r, :] for r in range(_NB))

        @plsc.parallel_loop(0, _JV, unroll=8, carry=w0)
        def col_body(jv, ws):
            sl = pl.ds(jv * _L, _L)
            i1 = c1_v[sl]
            i2 = c2_v[sl]
            bb1 = b1_v[sl]
            for r in range(_NB):
                av = abuf.at[s, r]
                bv = bbuf.at[s, r]
                a1 = plsc.load_gather(av, [i1])
                a2 = plsc.load_gather(av, [i2])
                g1 = plsc.load_gather(bv, [i1])
                g2 = plsc.load_gather(bv, [i2])
                t1 = a1 + ws[r] * (g1 - a1)
                t2 = a2 + ws[r] * (g2 - a2)
                obuf[s, r, sl] = t2 + bb1 * (t1 - t2)
            return ws

    stage(0, 0)
    stage(1, 1)

    def pair_body(k, carry):
        b0 = 2 * k
        gwait(0)

        @pl.when(k > 0)
        def _():
            swait(b0 - 2, 0)

        compute(b0, 0)
        store(b0, 0)
        stage(b0 + 2, 0)
        gwait(1)

        @pl.when(k > 0)
        def _():
            swait(b0 - 1, 1)

        compute(b0 + 1, 1)
        store(b0 + 1, 1)
        stage(jnp.minimum(b0 + 3, _NBLK - 1), 1)
        return carry

    lax.fori_loop(0, (_NBLK - 1) // 2, pair_body, 0)
    gwait(0)
    swait(_NBLK - 3, 0)
    compute(_NBLK - 1, 0)
    store(_NBLK - 1, 0)
    gwait(1)
    swait(_NBLK - 2, 1)
    swait(_NBLK - 1, 0)


_mesh = plsc.VectorSubcoreMesh(core_axis_name="c", subcore_axis_name="s")

_warp = functools.partial(
    pl.kernel,
    mesh=_mesh,
    compiler_params=pltpu.CompilerParams(
        use_tc_tiling_on_sc=False, needs_layout_passes=False),
    out_type=jax.ShapeDtypeStruct((_ROWS, _W), jnp.float32),
    scratch_types=[
        pltpu.VMEM((_RPW,), jnp.int32),
        pltpu.VMEM((_RPW,), jnp.int32),
        pltpu.VMEM((_RPW, _L), jnp.float32),
        pltpu.VMEM((2, _NB, _W), jnp.float32),
        pltpu.VMEM((2, _NB, _W), jnp.float32),
        pltpu.VMEM((2, _NB, _W), jnp.float32),
        pltpu.VMEM((_W,), jnp.int32),
        pltpu.VMEM((_W,), jnp.int32),
        pltpu.VMEM((_W,), jnp.float32),
        pltpu.SemaphoreType.DMA((2,)),
        pltpu.SemaphoreType.DMA((2,)),
        pltpu.SemaphoreType.DMA((2,)),
    ],
)(_sc_body)


def kernel(x, alpha):
    r1, r2, _, w2 = _coeffs(alpha, _YRES)
    c1, c2, b1, _ = _coeffs(alpha, _XRES)
    choff = (jnp.arange(_C, dtype=jnp.int32) * _H)[:, None]
    rows1 = (choff + r1[None, :]).reshape(-1)
    rows2 = (choff + r2[None, :]).reshape(-1)
    wfrep = jnp.broadcast_to(jnp.tile(w2, _C)[:, None], (_ROWS, _L))
    x2d = x.reshape(_ROWS, _W)
    out = _warp(x2d, rows1, rows2, wfrep, c1, c2, b1)
    return out.reshape(x.shape)


# speedup vs baseline: 1.0090x; 1.0090x over previous
import functools

import jax
import jax.numpy as jnp
from jax import lax
from jax.experimental import pallas as pl
from jax.experimental.pallas import tpu as pltpu
from jax.experimental.pallas import tpu_sc as plsc

_D = 9
_RAD = 4
_YRES = 256
_XRES = 256
_H = _YRES * _D
_W = _XRES * _D
_C = 3
_ROWS = _C * _H
_NW = 32
_RPW = _ROWS // _NW
_NB = 8
_NBLK = _RPW // _NB
_L = 16
_JV = _W // _L


def _coeffs(alpha, n_res):
    idx = jnp.arange(n_res * _D)
    mp = idx // _D
    off = idx % _D
    d = (off - _RAD).astype(jnp.float32)
    ind = mp.astype(jnp.float32) - alpha * d
    fl = jnp.floor(ind)
    w2 = ind - fl
    w1 = 1.0 - w2
    g1 = jnp.clip(fl, 0, n_res - 1).astype(jnp.int32)
    g2 = jnp.clip(fl + 1.0, 0, n_res - 1).astype(jnp.int32)
    r1 = off + g1 * _D
    r2 = off + g2 * _D
    return r1, r2, w1, w2


def _sc_body(x_hbm, rows1_hbm, rows2_hbm, wf_hbm,
             c1_hbm, c2_hbm, b1_hbm, out_hbm,
             idx1_v, idx2_v, wf_v, abuf, bbuf, obuf,
             c1_v, c2_v, b1_v, sem_a, sem_b, sem_o):
    wid = lax.axis_index("s") * 2 + lax.axis_index("c")
    my_base = wid * _RPW

    pltpu.sync_copy(c1_hbm, c1_v)
    pltpu.sync_copy(c2_hbm, c2_v)
    pltpu.sync_copy(b1_hbm, b1_v)
    pltpu.sync_copy(rows1_hbm.at[pl.ds(my_base, _RPW)], idx1_v)
    pltpu.sync_copy(rows2_hbm.at[pl.ds(my_base, _RPW)], idx2_v)
    pltpu.sync_copy(wf_hbm.at[pl.ds(my_base, _RPW)], wf_v)

    def stage(blk, s):
        ofs = blk * _NB
        pltpu.async_copy(
            x_hbm.at[idx1_v.at[pl.ds(ofs, _NB)]], abuf.at[s], sem_a.at[s])
        pltpu.async_copy(
            x_hbm.at[idx2_v.at[pl.ds(ofs, _NB)]], bbuf.at[s], sem_b.at[s])

    def gwait(s):
        pltpu.make_async_copy(
            x_hbm.at[idx1_v.at[pl.ds(0, _NB)]], abuf.at[s], sem_a.at[s]).wait()
        pltpu.make_async_copy(
            x_hbm.at[idx2_v.at[pl.ds(0, _NB)]], bbuf.at[s], sem_b.at[s]).wait()

    def store(blk, s):
        base = blk * _NB + my_base
        pltpu.async_copy(
            obuf.at[s], out_hbm.at[pl.ds(base, _NB)], sem_o.at[s])

    def swait(blk, s):
        base = blk * _NB + my_base
        pltpu.make_async_copy(
            obuf.at[s], out_hbm.at[pl.ds(base, _NB)], sem_o.at[s]).wait()

    def compute(blk, s):
        ofs = blk * _NB
        w0 = tuple(wf_v[ofs + r, :] for r in range(_NB))

        @plsc.parallel_loop(0, _JV, unroll=4, carry=w0)
        def col_body(jv, ws):
            sl = pl.ds(jv * _L, _L)
            i1 = c1_v[sl]
            i2 = c2_v[sl]
            bb1 = b1_v[sl]
            for r in range(_NB):
                av = abuf.at[s, r]
                bv = bbuf.at[s, r]
                a1 = plsc.load_gather(av, [i1])
                a2 = plsc.load_gather(av, [i2])
                g1 = plsc.load_gather(bv, [i1])
                g2 = plsc.load_gather(bv, [i2])
                t1 = a1 + ws[r] * (g1 - a1)
                t2 = a2 + ws[r] * (g2 - a2)
                obuf[s, r, sl] = t2 + bb1 * (t1 - t2)
            return ws

    stage(0, 0)
    stage(1, 1)

    def pair_body(k, carry):
        b0 = 2 * k
        gwait(0)

        @pl.when(k > 0)
        def _():
            swait(b0 - 2, 0)

        compute(b0, 0)
        store(b0, 0)
        stage(b0 + 2, 0)
        gwait(1)

        @pl.when(k > 0)
        def _():
            swait(b0 - 1, 1)

        compute(b0 + 1, 1)
        store(b0 + 1, 1)
        stage(jnp.minimum(b0 + 3, _NBLK - 1), 1)
        return carry

    lax.fori_loop(0, (_NBLK - 1) // 2, pair_body, 0)
    gwait(0)
    swait(_NBLK - 3, 0)
    compute(_NBLK - 1, 0)
    store(_NBLK - 1, 0)
    gwait(1)
    swait(_NBLK - 2, 1)
    swait(_NBLK - 1, 0)


_mesh = plsc.VectorSubcoreMesh(core_axis_name="c", subcore_axis_name="s")

_warp = functools.partial(
    pl.kernel,
    mesh=_mesh,
    compiler_params=pltpu.CompilerParams(
        use_tc_tiling_on_sc=False, needs_layout_passes=False),
    out_type=jax.ShapeDtypeStruct((_ROWS, _W), jnp.float32),
    scratch_types=[
        pltpu.VMEM((_RPW,), jnp.int32),
        pltpu.VMEM((_RPW,), jnp.int32),
        pltpu.VMEM((_RPW, _L), jnp.float32),
        pltpu.VMEM((2, _NB, _W), jnp.float32),
        pltpu.VMEM((2, _NB, _W), jnp.float32),
        pltpu.VMEM((2, _NB, _W), jnp.float32),
        pltpu.VMEM((_W,), jnp.int32),
        pltpu.VMEM((_W,), jnp.int32),
        pltpu.VMEM((_W,), jnp.float32),
        pltpu.SemaphoreType.DMA((2,)),
        pltpu.SemaphoreType.DMA((2,)),
        pltpu.SemaphoreType.DMA((2,)),
    ],
)(_sc_body)


def kernel(x, alpha):
    r1, r2, _, w2 = _coeffs(alpha, _YRES)
    c1, c2, b1, _ = _coeffs(alpha, _XRES)
    choff = (jnp.arange(_C, dtype=jnp.int32) * _H)[:, None]
    rows1 = (choff + r1[None, :]).reshape(-1)
    rows2 = (choff + r2[None, :]).reshape(-1)
    wfrep = jnp.broadcast_to(jnp.tile(w2, _C)[:, None], (_ROWS, _L))
    x2d = x.reshape(_ROWS, _W)
    out = _warp(x2d, rows1, rows2, wfrep, c1, c2, b1)
    return out.reshape(x.shape)
